# baseline (speedup 1.0000x reference)
import functools

import jax
import jax.numpy as jnp
from jax import lax
from jax.experimental import pallas as pl
from jax.experimental.pallas import tpu as pltpu
from jax.experimental.pallas import tpu_sc as plsc

NC = 2
NS = 16
L = 16
NW = NC * NS

BATCH = 16384
DIM = 64
BPW = BATCH // NW
CHUNK = 128
NCHUNK = BPW // CHUNK


def _gmf_body(user_table, item_table, user, item, out,
              uidx_v, iidx_v, urows_v, irows_v, out_v, sem):
    wid = lax.axis_index("s") * NC + lax.axis_index("c")
    base = wid * BPW

    pltpu.sync_copy(user.at[pl.ds(base, BPW)], uidx_v)
    pltpu.sync_copy(item.at[pl.ds(base, BPW)], iidx_v)

    copies = []
    for j in range(NCHUNK):
        idx = uidx_v.at[pl.ds(j * CHUNK, CHUNK)]
        dst = urows_v.at[pl.ds(j * CHUNK, CHUNK)]
        copies.append(pltpu.async_copy(user_table.at[idx], dst, sem))
    for j in range(NCHUNK):
        idx = iidx_v.at[pl.ds(j * CHUNK, CHUNK)]
        dst = irows_v.at[pl.ds(j * CHUNK, CHUNK)]
        copies.append(pltpu.async_copy(item_table.at[idx], dst, sem))
    for c in copies:
        c.wait()

    lane = lax.iota(jnp.int32, L)

    def group(g, carry):
        def elem(k, r):
            e = g * L + k
            p = (urows_v[e, pl.ds(0, 16)] * irows_v[e, pl.ds(0, 16)]
                 + urows_v[e, pl.ds(16, 16)] * irows_v[e, pl.ds(16, 16)]
                 + urows_v[e, pl.ds(32, 16)] * irows_v[e, pl.ds(32, 16)]
                 + urows_v[e, pl.ds(48, 16)] * irows_v[e, pl.ds(48, 16)])
            s = jnp.sum(p)
            return jnp.where(lane == k, s, r)

        r = lax.fori_loop(0, L, elem, jnp.zeros((L,), jnp.float32))
        out_v[pl.ds(g * L, L)] = 1.0 / (1.0 + jnp.exp(-r))
        return carry

    lax.fori_loop(0, BPW // L, group, 0)

    pltpu.sync_copy(out_v, out.at[pl.ds(base, BPW)])


_gmf = functools.partial(
    pl.kernel,
    out_type=jax.ShapeDtypeStruct((BATCH,), jnp.float32),
    mesh=plsc.VectorSubcoreMesh(core_axis_name="c", subcore_axis_name="s"),
    scratch_types=[
        pltpu.VMEM((BPW,), jnp.int32),
        pltpu.VMEM((BPW,), jnp.int32),
        pltpu.VMEM((BPW, DIM), jnp.float32),
        pltpu.VMEM((BPW, DIM), jnp.float32),
        pltpu.VMEM((BPW,), jnp.float32),
        pltpu.SemaphoreType.DMA,
    ],
    compiler_params=pltpu.CompilerParams(
        needs_layout_passes=False, use_tc_tiling_on_sc=False),
)(_gmf_body)


def kernel(user_table, item_table, user, item):
    return _gmf(user_table, item_table,
                user.astype(jnp.int32), item.astype(jnp.int32))
